# Initial kernel scaffold; baseline (speedup 1.0000x reference)
import jax
import jax.numpy as jnp
from jax.experimental import pallas as pl


def kernel(input_ids, token_table, pe):
    raise NotImplementedError("write your pallas kernel here")


# speedup vs baseline: 4.6640x; 4.6640x over previous
import functools

import jax
import jax.numpy as jnp
from jax import lax
from jax.experimental import pallas as pl
from jax.experimental.pallas import tpu as pltpu
from jax.experimental.pallas import tpu_sc as plsc

B, S, D, V = 1024, 200, 64, 100000
NC, NS = 2, 16
NW = NC * NS
N = B * S
PER_W = N // NW
CHUNK = 128
NCHUNK = PER_W // CHUNK
NBUF = 5
NT = NCHUNK // NBUF

_mesh = plsc.VectorSubcoreMesh(core_axis_name="c", subcore_axis_name="s")


@functools.partial(
    pl.kernel,
    mesh=_mesh,
    out_type=jax.ShapeDtypeStruct((N, D), jnp.float32),
    compiler_params=pltpu.CompilerParams(use_tc_tiling_on_sc=False),
    scratch_types=(
        [pltpu.VMEM((PER_W,), jnp.int32),
         pltpu.VMEM((2 * S, D), jnp.float32)]
        + [pltpu.VMEM((CHUNK, D), jnp.float32) for _ in range(NBUF)]
        + [pltpu.SemaphoreType.DMA for _ in range(2 * NBUF)]
    ),
)
def _embed(ids_hbm, table_hbm, pe2_hbm, out_hbm, idx_v, pe_v, *rest):
    bufs = list(rest[:NBUF])
    gsem = list(rest[NBUF:2 * NBUF])
    osem = list(rest[2 * NBUF:3 * NBUF])
    wid = lax.axis_index("s") * NC + lax.axis_index("c")

    pltpu.sync_copy(ids_hbm.at[pl.ds(wid * PER_W, PER_W)], idx_v)
    pltpu.sync_copy(pe2_hbm, pe_v)
    out_base = wid * PER_W

    def fire_gather(c, slot):
        pltpu.async_copy(table_hbm.at[idx_v.at[pl.ds(c * CHUNK, CHUNK)]],
                         bufs[slot], gsem[slot])

    def wait_gather(c, slot):
        pltpu.make_async_copy(
            table_hbm.at[idx_v.at[pl.ds(c * CHUNK, CHUNK)]],
            bufs[slot], gsem[slot]).wait()

    def fire_out(c, slot):
        pltpu.async_copy(
            bufs[slot], out_hbm.at[pl.ds(out_base + c * CHUNK, CHUNK)],
            osem[slot])

    def wait_out(slot):
        pltpu.make_async_copy(
            bufs[slot], out_hbm.at[pl.ds(out_base, CHUNK)],
            osem[slot]).wait()

    def add_pe(c, slot):
        buf = bufs[slot]
        off = lax.rem(c * CHUNK, S)

        @pl.loop(0, CHUNK, step=4)
        def _(r):
            for dr in range(4):
                for k in range(4):
                    sl = pl.ds(k * 16, 16)
                    plsc.addupdate(buf.at[r + dr, sl],
                                   pe_v[off + r + dr, sl])

    def do_chunk(c, b, first, last):
        ps = (b + NBUF - 1) % NBUF
        if not first:
            wait_out(ps)
        if not last:
            fire_gather(c + NBUF - 1, ps)
        wait_gather(c, b)
        add_pe(c, b)
        fire_out(c, b)

    for c0 in range(NBUF - 1):
        fire_gather(c0, c0)

    for b in range(NBUF):
        do_chunk(b, b, first=(b == 0), last=False)

    @pl.loop(1, NT - 1)
    def _(t):
        for b in range(NBUF):
            do_chunk(t * NBUF + b, b, first=False, last=False)

    for b in range(NBUF):
        c = (NT - 1) * NBUF + b
        do_chunk(c, b, first=False, last=(c + NBUF - 1 >= NCHUNK))

    wait_out((NCHUNK - 1) % NBUF)


def kernel(input_ids, token_table, pe):
    ids_flat = input_ids.reshape(N)
    pe2 = jnp.concatenate([pe[:S], pe[:S]], axis=0)
    out = _embed(ids_flat, token_table, pe2)
    return out.reshape(B, S, D)


# speedup vs baseline: 6.7281x; 1.4426x over previous
import functools

import jax
import jax.numpy as jnp
from jax import lax
from jax.experimental import pallas as pl
from jax.experimental.pallas import tpu as pltpu
from jax.experimental.pallas import tpu_sc as plsc

B, S, D, V = 1024, 200, 64, 100000
NC, NS = 2, 16
LANE = 128
NBR = B // LANE
NBPC = D // 8 // NC
NROUND = 2
SCH = 40
NSCH = S // SCH
NBUF = 2
NT = SCH // NBUF

_mesh = plsc.VectorSubcoreMesh(core_axis_name="c", subcore_axis_name="s")


@functools.partial(
    pl.kernel,
    mesh=_mesh,
    out_type=jax.ShapeDtypeStruct((S, D // 8, NBR, 8, LANE), jnp.float32),
    compiler_params=pltpu.CompilerParams(use_tc_tiling_on_sc=False,
                                         needs_layout_passes=False),
    scratch_types=(
        [pltpu.VMEM_SHARED((2, 8, V), jnp.float32),
         pltpu.VMEM((SCH, LANE), jnp.int32),
         pltpu.VMEM((8, 512), jnp.float32)]
        + [pltpu.VMEM((8, LANE), jnp.float32) for _ in range(NBUF)]
        + [pltpu.SemaphoreType.DMA for _ in range(2 * NBUF + 1)]
    ),
)
def _embedT(ids_hbm, table_hbm, pe_hbm, out_hbm, spm, ids_v, pe_v, *rest):
    blk = list(rest[:NBUF])
    gsem = list(rest[NBUF:2 * NBUF])
    osem = list(rest[2 * NBUF:3 * NBUF])
    ssem = rest[3 * NBUF]
    core = lax.axis_index("c")
    tid = lax.axis_index("s")
    bl = tid // NBR
    bt = tid % NBR

    def run_round(r):
        band = core * NBPC + r * NROUND + bl
        @pl.when(bt == 0)
        def _():
            pltpu.async_copy(table_hbm.at[pl.ds(band * 8, 8)],
                             spm.at[bl], ssem).wait()
        pltpu.sync_copy(pe_hbm.at[pl.ds(band * 8, 8)], pe_v)
        plsc.subcore_barrier()

        def fire_gathers(s, j):
            for dloc in range(8):
                pltpu.async_copy(spm.at[bl, dloc].at[ids_v.at[s]],
                                 blk[j].at[dloc], gsem[j])

        def wait_gathers(s, j):
            for dloc in range(8):
                pltpu.make_async_copy(spm.at[bl, dloc].at[ids_v.at[s]],
                                      blk[j].at[dloc], gsem[j]).wait()

        def fire_out(sg, j):
            pltpu.async_copy(blk[j], out_hbm.at[sg, band, bt], osem[j])

        def wait_out(j):
            pltpu.make_async_copy(blk[j], out_hbm.at[0, 0, 0],
                                  osem[j]).wait()

        def add_pe(sg, j):
            s16 = (sg // 16) * 16
            onehot = lax.iota(jnp.int32, 16) == (sg - s16)
            for dloc in range(8):
                v16 = pe_v[dloc, pl.ds(s16, 16)]
                val = jnp.sum(jnp.where(onehot, v16, 0.0))
                bvec = jnp.full((16,), val, jnp.float32)
                for k in range(LANE // 16):
                    plsc.addupdate(blk[j].at[dloc, pl.ds(k * 16, 16)], bvec)

        def do_pos(s0, i, j, first, last):
            ps = (j + NBUF - 1) % NBUF
            if not first:
                wait_out(ps)
            if not last:
                fire_gathers(i + 1, ps)
            wait_gathers(i, j)
            add_pe(s0 + i, j)
            fire_out(s0 + i, j)

        @pl.loop(0, NSCH)
        def _(sc):
            s0 = sc * SCH
            pltpu.sync_copy(
                ids_hbm.at[pl.ds(s0, SCH), pl.ds(bt * LANE, LANE)], ids_v)
            fire_gathers(0, 0)
            for b in range(NBUF):
                do_pos(s0, b, b, first=(b == 0), last=False)

            @pl.loop(1, NT - 1)
            def _(t):
                for b in range(NBUF):
                    do_pos(s0, t * NBUF + b, b, first=False, last=False)

            for b in range(NBUF):
                i = (NT - 1) * NBUF + b
                do_pos(s0, i, b, first=False, last=(i + 1 >= SCH))
            wait_out((SCH - 1) % NBUF)

        plsc.subcore_barrier()

    run_round(0)
    run_round(1)


def kernel(input_ids, token_table, pe):
    out5 = _embedT(input_ids.T, token_table.T, pe.T)
    return jnp.transpose(out5, (2, 4, 0, 1, 3)).reshape(B, S, D)
